# baseline (speedup 1.0000x reference)
import functools

import jax
import jax.numpy as jnp
from jax import lax
from jax.experimental import pallas as pl
from jax.experimental.pallas import tpu as pltpu
from jax.experimental.pallas import tpu_sc as plsc

D_MODEL = 32
_NC = 2
_NS = 16
_NW = _NC * _NS
_CHUNK = 128
_NBUF = 4
_L = 16


@functools.cache
def _make_gather(N_I: int, N_J: int):
    i_per_w = N_I // _NW
    n_chunks_j = i_per_w // _CHUNK
    n_chunks = N_J * n_chunks_j
    n_groups = n_chunks // _NBUF
    mesh = plsc.VectorSubcoreMesh(core_axis_name="c", subcore_axis_name="s")

    @functools.partial(
        pl.kernel,
        mesh=mesh,
        out_type=jax.ShapeDtypeStruct((N_J, D_MODEL, N_I), jnp.float32),
        scratch_types=[
            pltpu.VMEM((N_J * i_per_w,), jnp.int32),
            pltpu.VMEM((N_J, i_per_w), jnp.int32),
            [pltpu.VMEM((_CHUNK, D_MODEL), jnp.float32) for _ in range(_NBUF)],
            [pltpu.VMEM((D_MODEL, _CHUNK), jnp.float32) for _ in range(_NBUF)],
            [pltpu.SemaphoreType.DMA for _ in range(_NBUF)],
            [pltpu.SemaphoreType.DMA for _ in range(_NBUF)],
        ],
        compiler_params=pltpu.CompilerParams(
            use_tc_tiling_on_sc=False, needs_layout_passes=False),
    )
    def gather(table_hbm, idx_hbm, out_hbm, x_v, idx_v, rows, trans, gsem, wsem):
        wid = lax.axis_index("s") * _NC + lax.axis_index("c")
        i0 = wid * i_per_w
        pltpu.sync_copy(idx_hbm.at[pl.ds(i0 * N_J, i_per_w * N_J)], x_v)
        iota = lax.iota(jnp.int32, _L)
        iota_nj = iota * N_J

        def jbody(j, carry):
            for l0 in range(0, i_per_w, _L):
                addr = iota_nj + (l0 * N_J + j)
                v = plsc.load_gather(x_v, [addr])
                idx_v[j, pl.ds(l0, _L)] = v
            return carry

        lax.fori_loop(0, N_J, jbody, 0)

        def start_gather(c, b):
            j = c // n_chunks_j
            col = (c % n_chunks_j) * _CHUNK
            pltpu.async_copy(
                table_hbm.at[idx_v.at[j, pl.ds(col, _CHUNK)]],
                rows[b], gsem[b])

        def wait_gather(b):
            pltpu.make_async_copy(
                table_hbm.at[pl.ds(0, _CHUNK)], rows[b], gsem[b]).wait()

        def start_wb(c, b):
            j = c // n_chunks_j
            col = i0 + (c % n_chunks_j) * _CHUNK
            pltpu.async_copy(
                trans[b], out_hbm.at[j, :, pl.ds(col, _CHUNK)], wsem[b])

        def wait_wb(b):
            pltpu.make_async_copy(
                trans[b], out_hbm.at[0, :, pl.ds(0, _CHUNK)], wsem[b]).wait()

        def transpose(b):
            iota = lax.iota(jnp.int32, _L)

            def kbody(k, carry):
                colbase = (iota + k) & (_L - 1)
                for l0 in range(0, _CHUNK, _L):
                    row = iota + l0
                    for d0 in range(0, D_MODEL, _L):
                        col = colbase + d0
                        v = plsc.load_gather(rows[b], [row, col])
                        plsc.store_scatter(trans[b], [col, row], v)
                return carry

            lax.fori_loop(0, _L, kbody, 0)

        for b in range(_NBUF):
            start_gather(b, b)

        def body(g, carry):
            for b in range(_NBUF):
                c = g * _NBUF + b
                wait_gather(b)

                @pl.when(g != 0)
                def _():
                    wait_wb(b)

                transpose(b)
                start_wb(c, b)

                @pl.when(g < n_groups - 1)
                def _():
                    start_gather(c + _NBUF, b)

            return carry

        lax.fori_loop(0, n_groups, body, 0)
        for b in range(_NBUF):
            wait_wb(b)

    return gather


def kernel(x, W):
    n_b, n_h = x.shape
    xf = x.astype(jnp.int32).reshape(-1)
    out_t = _make_gather(n_b, n_h)(W, xf)
    return jnp.transpose(out_t, (2, 0, 1))
